# Initial kernel scaffold; baseline (speedup 1.0000x reference)
import jax
import jax.numpy as jnp
from jax.experimental import pallas as pl


def kernel(x, edge_index, batch, params):
    raise NotImplementedError("write your pallas kernel here")


# speedup vs baseline: 7.9154x; 7.9154x over previous
import functools

import jax
import jax.numpy as jnp
from jax import lax
from jax.experimental import pallas as pl
from jax.experimental.pallas import tpu as pltpu
from jax.experimental.pallas import tpu_sc as plsc

_N = 10000
_E = 320000
_F = 128
_H = 32
_C = 10
_G = 64

_NC = 2
_NS = 16
_NW = _NC * _NS

_EB = 128
_GPW = 80
_EPAD = _NW * _GPW * _EB
_GBLK = 8
_NBLK = _GPW // _GBLK
_NPAD = 10240
_RPT = _NPAD // _NS


def _edge_mesh():
    return plsc.VectorSubcoreMesh(
        core_axis_name="c", subcore_axis_name="s",
        num_cores=_NC, num_subcores=_NS)


@functools.partial(
    pl.kernel,
    out_type=jax.ShapeDtypeStruct((_NC * _NPAD, _H), jnp.float32),
    mesh=_edge_mesh(),
    scratch_types=[
        pltpu.VMEM((_GBLK, _EB), jnp.int32),
        pltpu.VMEM((_GBLK, _EB), jnp.int32),
        pltpu.VMEM((_GBLK, _EB, _H), jnp.float32),
        pltpu.VMEM((_EB, _H), jnp.float32),
        pltpu.VMEM_SHARED((_NPAD, _H), jnp.float32),
        pltpu.SemaphoreType.DMA,
    ],
    compiler_params=pltpu.CompilerParams(use_tc_tiling_on_sc=False),
)
def _edge_agg(src_hbm, dst_hbm, hp_hbm, out_hbm,
              src_v, dst_v, rows_v, zero_v, acc_sh, sem):
    cid = lax.axis_index("c")
    sid = lax.axis_index("s")
    wid = cid * _NS + sid

    def _zrow(i, _):
        z16 = jnp.zeros((16,), jnp.float32)
        zero_v[i, 0:16] = z16
        zero_v[i, 16:32] = z16
        return 0
    lax.fori_loop(0, _EB, _zrow, 0)
    for k in range(_RPT // _EB):
        pltpu.sync_copy(zero_v, acc_sh.at[pl.ds(sid * _RPT + k * _EB, _EB)])
    plsc.subcore_barrier()

    def _block(b, _):
        g0 = wid * _GPW + b * _GBLK
        pltpu.sync_copy(src_hbm.at[pl.ds(g0, _GBLK)], src_v)
        pltpu.sync_copy(dst_hbm.at[pl.ds(g0, _GBLK)], dst_v)
        descs = [
            pltpu.async_copy(hp_hbm.at[src_v.at[j]], rows_v.at[j], sem)
            for j in range(_GBLK)
        ]
        for d in descs:
            d.wait()
        for j in range(_GBLK):
            pltpu.sync_copy(rows_v.at[j], acc_sh.at[dst_v.at[j]], add=True)
        return 0
    lax.fori_loop(0, _NBLK, _block, 0)

    plsc.subcore_barrier()
    base = cid * _NPAD + sid * _RPT
    for k in range(_RPT // _EB):
        pltpu.sync_copy(acc_sh.at[pl.ds(sid * _RPT + k * _EB, _EB)],
                        out_hbm.at[pl.ds(base + k * _EB, _EB)])


def _proj_body(x_ref, w_ref, o_ref):
    o_ref[...] = jnp.dot(x_ref[...], w_ref[...],
                         preferred_element_type=jnp.float32)


def _mid_body(hp_ref, parts_ref, b1_ref, gamma_ref, beta_ref,
              w2_ref, b2_ref, w1n_ref, o_ref):
    z = (hp_ref[...] + parts_ref[0:_N, :] + parts_ref[_NPAD:_NPAD + _N, :]
         + b1_ref[...])
    mean = jnp.mean(z, axis=0, keepdims=True)
    var = jnp.mean((z - mean) ** 2, axis=0, keepdims=True)
    zn = (z - mean) * lax.rsqrt(var + 1e-5) * gamma_ref[...] + beta_ref[...]
    a = jnp.maximum(zn, 0.0)
    h = jnp.maximum(
        jnp.dot(a, w2_ref[...], preferred_element_type=jnp.float32)
        + b2_ref[...], 0.0)
    o_ref[...] = jnp.dot(h, w1n_ref[...], preferred_element_type=jnp.float32)


def _last_body(hp_ref, parts_ref, b1_ref, gamma_ref, beta_ref,
               w2_ref, b2_ref, batch_ref, fw1_ref, fb1_ref,
               fw2_ref, fb2_ref, o_ref):
    z = (hp_ref[...] + parts_ref[0:_N, :] + parts_ref[_NPAD:_NPAD + _N, :]
         + b1_ref[...])
    mean = jnp.mean(z, axis=0, keepdims=True)
    var = jnp.mean((z - mean) ** 2, axis=0, keepdims=True)
    zn = (z - mean) * lax.rsqrt(var + 1e-5) * gamma_ref[...] + beta_ref[...]
    a = jnp.maximum(zn, 0.0)
    h = jnp.maximum(
        jnp.dot(a, w2_ref[...], preferred_element_type=jnp.float32)
        + b2_ref[...], 0.0)
    giota = lax.broadcasted_iota(jnp.int32, (_N, _G), 1)
    onehot = (batch_ref[...] == giota).astype(jnp.float32)
    g = lax.dot_general(onehot, h, (((0,), (0,)), ((), ())),
                        preferred_element_type=jnp.float32)
    g = jnp.maximum(
        jnp.dot(g, fw1_ref[...], preferred_element_type=jnp.float32)
        + fb1_ref[...], 0.0)
    o_ref[...] = (jnp.dot(g, fw2_ref[...], preferred_element_type=jnp.float32)
                  + fb2_ref[...])


_proj = pl.pallas_call(
    _proj_body, out_shape=jax.ShapeDtypeStruct((_N, _H), jnp.float32))

_mid = pl.pallas_call(
    _mid_body, out_shape=jax.ShapeDtypeStruct((_N, _H), jnp.float32))

_last = pl.pallas_call(
    _last_body, out_shape=jax.ShapeDtypeStruct((_G, _C), jnp.float32))


def kernel(x, edge_index, batch, params):
    src = edge_index[0]
    dst = edge_index[1]
    pad = _EPAD - _E
    src_p = jnp.concatenate(
        [src, jnp.zeros((pad,), jnp.int32)]).reshape(_EPAD // _EB, _EB)
    dst_p = jnp.concatenate(
        [dst, jnp.full((pad,), _N, jnp.int32)]).reshape(_EPAD // _EB, _EB)

    convs = params['convs']
    row = lambda v: v.reshape(1, -1)

    hp = _proj(x, convs[0]['W1'])
    for l in range(4):
        p = convs[l]
        parts = _edge_agg(src_p, dst_p, hp)
        hp = _mid(hp, parts, row(p['b1']), row(p['gamma']), row(p['beta']),
                  p['W2'], row(p['b2']), convs[l + 1]['W1'])
    p = convs[4]
    parts = _edge_agg(src_p, dst_p, hp)
    out = _last(hp, parts, row(p['b1']), row(p['gamma']), row(p['beta']),
                p['W2'], row(p['b2']), batch.reshape(_N, 1),
                params['fW1'], row(params['fb1']),
                params['fW2'], row(params['fb2']))
    return out


# speedup vs baseline: 7.9184x; 1.0004x over previous
import functools

import jax
import jax.numpy as jnp
from jax import lax
from jax.experimental import pallas as pl
from jax.experimental.pallas import tpu as pltpu
from jax.experimental.pallas import tpu_sc as plsc

_N = 10000
_E = 320000
_F = 128
_H = 32
_C = 10
_G = 64

_NC = 2
_NS = 16
_NW = _NC * _NS

_EB = 128
_GPW = 80
_EPAD = _NW * _GPW * _EB
_GBLK = 8
_NBLK = _GPW // _GBLK
_NPAD = 10240
_RPT = _NPAD // _NS


def _edge_mesh():
    return plsc.VectorSubcoreMesh(
        core_axis_name="c", subcore_axis_name="s",
        num_cores=_NC, num_subcores=_NS)


@functools.partial(
    pl.kernel,
    out_type=jax.ShapeDtypeStruct((_NC * _NPAD, _H), jnp.float32),
    mesh=_edge_mesh(),
    scratch_types=[
        pltpu.VMEM((_GBLK, _EB), jnp.int32),
        pltpu.VMEM((_GBLK, _EB), jnp.int32),
        pltpu.VMEM((_GBLK, _EB, _H), jnp.float32),
        pltpu.VMEM((_EB, _H), jnp.float32),
        pltpu.VMEM_SHARED((_NPAD, _H), jnp.float32),
        pltpu.SemaphoreType.DMA,
    ],
    compiler_params=pltpu.CompilerParams(use_tc_tiling_on_sc=False),
)
def _edge_agg(src_hbm, dst_hbm, hp_hbm, out_hbm,
              src_v, dst_v, rows_v, zero_v, acc_sh, sem):
    cid = lax.axis_index("c")
    sid = lax.axis_index("s")
    wid = cid * _NS + sid

    def _zrow(i, _):
        z16 = jnp.zeros((16,), jnp.float32)
        zero_v[i, 0:16] = z16
        zero_v[i, 16:32] = z16
        return 0
    lax.fori_loop(0, _EB, _zrow, 0)
    for k in range(_RPT // _EB):
        pltpu.sync_copy(zero_v, acc_sh.at[pl.ds(sid * _RPT + k * _EB, _EB)])
    plsc.subcore_barrier()

    def _block(b, _):
        g0 = wid * _GPW + b * _GBLK
        pltpu.sync_copy(src_hbm.at[pl.ds(g0, _GBLK)], src_v)
        pltpu.sync_copy(dst_hbm.at[pl.ds(g0, _GBLK)], dst_v)
        descs = [
            pltpu.async_copy(hp_hbm.at[src_v.at[j]], rows_v.at[j], sem)
            for j in range(_GBLK)
        ]
        for d in descs:
            d.wait()
        for j in range(_GBLK):
            pltpu.sync_copy(rows_v.at[j], acc_sh.at[dst_v.at[j]], add=True)
        return 0
    lax.fori_loop(0, _NBLK, _block, 0)

    plsc.subcore_barrier()
    base = cid * _NPAD + sid * _RPT
    for k in range(_RPT // _EB):
        pltpu.sync_copy(acc_sh.at[pl.ds(sid * _RPT + k * _EB, _EB)],
                        out_hbm.at[pl.ds(base + k * _EB, _EB)])


def _proj_body(x_ref, w_ref, o_ref):
    o_ref[...] = jnp.dot(x_ref[...], w_ref[...],
                         preferred_element_type=jnp.float32)


def _mid_body(hp_ref, parts_ref, b1_ref, gamma_ref, beta_ref,
              w2_ref, b2_ref, w1n_ref, o_ref):
    z = (hp_ref[...] + parts_ref[0:_N, :] + parts_ref[_NPAD:_NPAD + _N, :]
         + b1_ref[...])
    mean = jnp.mean(z, axis=0, keepdims=True)
    var = jnp.mean((z - mean) ** 2, axis=0, keepdims=True)
    zn = (z - mean) * lax.rsqrt(var + 1e-5) * gamma_ref[...] + beta_ref[...]
    a = jnp.maximum(zn, 0.0)
    h = jnp.maximum(
        jnp.dot(a, w2_ref[...], preferred_element_type=jnp.float32)
        + b2_ref[...], 0.0)
    o_ref[...] = jnp.dot(h, w1n_ref[...], preferred_element_type=jnp.float32)


def _last_body(hp_ref, parts_ref, b1_ref, gamma_ref, beta_ref,
               w2_ref, b2_ref, batch_ref, fw1_ref, fb1_ref,
               fw2_ref, fb2_ref, o_ref):
    z = (hp_ref[...] + parts_ref[0:_N, :] + parts_ref[_NPAD:_NPAD + _N, :]
         + b1_ref[...])
    mean = jnp.mean(z, axis=0, keepdims=True)
    var = jnp.mean((z - mean) ** 2, axis=0, keepdims=True)
    zn = (z - mean) * lax.rsqrt(var + 1e-5) * gamma_ref[...] + beta_ref[...]
    a = jnp.maximum(zn, 0.0)
    h = jnp.maximum(
        jnp.dot(a, w2_ref[...], preferred_element_type=jnp.float32)
        + b2_ref[...], 0.0)
    giota = lax.broadcasted_iota(jnp.int32, (_N, _G), 1)
    onehot = (batch_ref[...] == giota).astype(jnp.float32)
    g = lax.dot_general(onehot, h, (((0,), (0,)), ((), ())),
                        preferred_element_type=jnp.float32)
    g = jnp.maximum(
        jnp.dot(g, fw1_ref[...], preferred_element_type=jnp.float32)
        + fb1_ref[...], 0.0)
    o_ref[...] = (jnp.dot(g, fw2_ref[...], preferred_element_type=jnp.float32)
                  + fb2_ref[...])


_proj = pl.pallas_call(
    _proj_body, out_shape=jax.ShapeDtypeStruct((_N, _H), jnp.float32))

_mid = pl.pallas_call(
    _mid_body, out_shape=jax.ShapeDtypeStruct((_N, _H), jnp.float32))

_last = pl.pallas_call(
    _last_body, out_shape=jax.ShapeDtypeStruct((_G, _C), jnp.float32))


def kernel(x, edge_index, batch, params):
    src = edge_index[0]
    dst = edge_index[1]
    pad = _EPAD - _E
    src_p = jnp.concatenate(
        [src, jnp.zeros((pad,), jnp.int32)]).reshape(_EPAD // _EB, _EB)
    dst_pad = _N + (jnp.arange(pad, dtype=jnp.int32) % (_NPAD - _N))
    dst_p = jnp.concatenate([dst, dst_pad]).reshape(_EPAD // _EB, _EB)

    convs = params['convs']
    row = lambda v: v.reshape(1, -1)

    hp = _proj(x, convs[0]['W1'])
    for l in range(4):
        p = convs[l]
        parts = _edge_agg(src_p, dst_p, hp)
        hp = _mid(hp, parts, row(p['b1']), row(p['gamma']), row(p['beta']),
                  p['W2'], row(p['b2']), convs[l + 1]['W1'])
    p = convs[4]
    parts = _edge_agg(src_p, dst_p, hp)
    out = _last(hp, parts, row(p['b1']), row(p['gamma']), row(p['beta']),
                p['W2'], row(p['b2']), batch.reshape(_N, 1),
                params['fW1'], row(params['fb1']),
                params['fW2'], row(params['fb2']))
    return out


# speedup vs baseline: 8.9153x; 1.1259x over previous
import functools

import jax
import jax.numpy as jnp
from jax import lax
from jax.experimental import pallas as pl
from jax.experimental.pallas import tpu as pltpu
from jax.experimental.pallas import tpu_sc as plsc

_N = 10000
_E = 320000
_F = 128
_H = 32
_C = 10
_G = 64

_NC = 2
_NS = 16
_NW = _NC * _NS

_EB = 128
_GPW = 80
_EPAD = _NW * _GPW * _EB
_GBLK = 8
_NBLK = _GPW // _GBLK
_NPAD = 10240
_RPT = _NPAD // _NS


def _edge_mesh():
    return plsc.VectorSubcoreMesh(
        core_axis_name="c", subcore_axis_name="s",
        num_cores=_NC, num_subcores=_NS)


@functools.partial(
    pl.kernel,
    out_type=jax.ShapeDtypeStruct((_NC * _NPAD, _H), jnp.float32),
    mesh=_edge_mesh(),
    scratch_types=[
        pltpu.VMEM((3, _GBLK, _EB), jnp.int32),
        pltpu.VMEM((3, _GBLK, _EB), jnp.int32),
        pltpu.VMEM((2, _GBLK, _EB, _H), jnp.float32),
        pltpu.VMEM((_EB, _H), jnp.float32),
        pltpu.VMEM_SHARED((_NPAD, _H), jnp.float32),
        pltpu.SemaphoreType.DMA,
        pltpu.SemaphoreType.DMA,
        pltpu.SemaphoreType.DMA,
    ],
    compiler_params=pltpu.CompilerParams(use_tc_tiling_on_sc=False),
)
def _edge_agg(src_hbm, dst_hbm, hp_hbm, out_hbm,
              src_v, dst_v, rows_v, zero_v, acc_sh, sem_i, sem_g, sem_s):
    cid = lax.axis_index("c")
    sid = lax.axis_index("s")
    wid = cid * _NS + sid

    def _idx_load(b):
        g0 = wid * _GPW + b * _GBLK
        p = b % 3
        return [
            pltpu.async_copy(src_hbm.at[pl.ds(g0, _GBLK)], src_v.at[p], sem_i),
            pltpu.async_copy(dst_hbm.at[pl.ds(g0, _GBLK)], dst_v.at[p], sem_i),
        ]

    idx_d = {0: _idx_load(0)}

    def _zrow(i, _):
        z16 = jnp.zeros((16,), jnp.float32)
        zero_v[i, 0:16] = z16
        zero_v[i, 16:32] = z16
        return 0
    lax.fori_loop(0, _EB, _zrow, 0)
    for k in range(_RPT // _EB):
        pltpu.sync_copy(zero_v, acc_sh.at[pl.ds(sid * _RPT + k * _EB, _EB)])
    plsc.subcore_barrier()

    sca_d = {}
    for b in range(_NBLK):
        if b >= 2:
            for d in sca_d.pop(b - 2):
                d.wait()
        if b + 1 < _NBLK:
            idx_d[b + 1] = _idx_load(b + 1)
        for d in idx_d.pop(b):
            d.wait()
        p3, p2 = b % 3, b % 2
        gat = [
            pltpu.async_copy(hp_hbm.at[src_v.at[p3, j]], rows_v.at[p2, j],
                             sem_g)
            for j in range(_GBLK)
        ]
        for d in gat:
            d.wait()
        sca_d[b] = [
            pltpu.async_copy(rows_v.at[p2, j], acc_sh.at[dst_v.at[p3, j]],
                             sem_s, add=True)
            for j in range(_GBLK)
        ]
    for b in (_NBLK - 2, _NBLK - 1):
        for d in sca_d.pop(b):
            d.wait()

    plsc.subcore_barrier()
    base = cid * _NPAD + sid * _RPT
    for k in range(_RPT // _EB):
        pltpu.sync_copy(acc_sh.at[pl.ds(sid * _RPT + k * _EB, _EB)],
                        out_hbm.at[pl.ds(base + k * _EB, _EB)])


def _proj_body(x_ref, w_ref, o_ref):
    o_ref[...] = jnp.dot(x_ref[...], w_ref[...],
                         preferred_element_type=jnp.float32)


def _mid_body(hp_ref, parts_ref, b1_ref, gamma_ref, beta_ref,
              w2_ref, b2_ref, w1n_ref, o_ref):
    z = (hp_ref[...] + parts_ref[0:_N, :] + parts_ref[_NPAD:_NPAD + _N, :]
         + b1_ref[...])
    mean = jnp.mean(z, axis=0, keepdims=True)
    var = jnp.mean((z - mean) ** 2, axis=0, keepdims=True)
    zn = (z - mean) * lax.rsqrt(var + 1e-5) * gamma_ref[...] + beta_ref[...]
    a = jnp.maximum(zn, 0.0)
    h = jnp.maximum(
        jnp.dot(a, w2_ref[...], preferred_element_type=jnp.float32)
        + b2_ref[...], 0.0)
    o_ref[...] = jnp.dot(h, w1n_ref[...], preferred_element_type=jnp.float32)


def _last_body(hp_ref, parts_ref, b1_ref, gamma_ref, beta_ref,
               w2_ref, b2_ref, batch_ref, fw1_ref, fb1_ref,
               fw2_ref, fb2_ref, o_ref):
    z = (hp_ref[...] + parts_ref[0:_N, :] + parts_ref[_NPAD:_NPAD + _N, :]
         + b1_ref[...])
    mean = jnp.mean(z, axis=0, keepdims=True)
    var = jnp.mean((z - mean) ** 2, axis=0, keepdims=True)
    zn = (z - mean) * lax.rsqrt(var + 1e-5) * gamma_ref[...] + beta_ref[...]
    a = jnp.maximum(zn, 0.0)
    h = jnp.maximum(
        jnp.dot(a, w2_ref[...], preferred_element_type=jnp.float32)
        + b2_ref[...], 0.0)
    giota = lax.broadcasted_iota(jnp.int32, (_N, _G), 1)
    onehot = (batch_ref[...] == giota).astype(jnp.float32)
    g = lax.dot_general(onehot, h, (((0,), (0,)), ((), ())),
                        preferred_element_type=jnp.float32)
    g = jnp.maximum(
        jnp.dot(g, fw1_ref[...], preferred_element_type=jnp.float32)
        + fb1_ref[...], 0.0)
    o_ref[...] = (jnp.dot(g, fw2_ref[...], preferred_element_type=jnp.float32)
                  + fb2_ref[...])


_proj = pl.pallas_call(
    _proj_body, out_shape=jax.ShapeDtypeStruct((_N, _H), jnp.float32))

_mid = pl.pallas_call(
    _mid_body, out_shape=jax.ShapeDtypeStruct((_N, _H), jnp.float32))

_last = pl.pallas_call(
    _last_body, out_shape=jax.ShapeDtypeStruct((_G, _C), jnp.float32))


def kernel(x, edge_index, batch, params):
    src = edge_index[0]
    dst = edge_index[1]
    pad = _EPAD - _E
    src_p = jnp.concatenate(
        [src, jnp.zeros((pad,), jnp.int32)]).reshape(_EPAD // _EB, _EB)
    dst_pad = _N + (jnp.arange(pad, dtype=jnp.int32) % (_NPAD - _N))
    dst_p = jnp.concatenate([dst, dst_pad]).reshape(_EPAD // _EB, _EB)

    convs = params['convs']
    row = lambda v: v.reshape(1, -1)

    hp = _proj(x, convs[0]['W1'])
    for l in range(4):
        p = convs[l]
        parts = _edge_agg(src_p, dst_p, hp)
        hp = _mid(hp, parts, row(p['b1']), row(p['gamma']), row(p['beta']),
                  p['W2'], row(p['b2']), convs[l + 1]['W1'])
    p = convs[4]
    parts = _edge_agg(src_p, dst_p, hp)
    out = _last(hp, parts, row(p['b1']), row(p['gamma']), row(p['beta']),
                p['W2'], row(p['b2']), batch.reshape(_N, 1),
                params['fW1'], row(params['fb1']),
                params['fW2'], row(params['fb2']))
    return out


# speedup vs baseline: 9.0053x; 1.0101x over previous
import functools

import jax
import jax.numpy as jnp
from jax import lax
from jax.experimental import pallas as pl
from jax.experimental.pallas import tpu as pltpu
from jax.experimental.pallas import tpu_sc as plsc

_N = 10000
_E = 320000
_F = 128
_H = 32
_C = 10
_G = 64

_NC = 2
_NS = 16
_NW = _NC * _NS

_EB = 128
_GPW = 80
_EPAD = _NW * _GPW * _EB
_GBLK = 8
_NBLK = _GPW // _GBLK
_NPAD = 10240
_RPT = _NPAD // _NS


def _edge_mesh():
    return plsc.VectorSubcoreMesh(
        core_axis_name="c", subcore_axis_name="s",
        num_cores=_NC, num_subcores=_NS)


@functools.partial(
    pl.kernel,
    out_type=jax.ShapeDtypeStruct((_NC * _NPAD, _H), jnp.float32),
    mesh=_edge_mesh(),
    scratch_types=[
        pltpu.VMEM((3, _GBLK, _EB), jnp.int32),
        pltpu.VMEM((3, _GBLK, _EB), jnp.int32),
        pltpu.VMEM((2, _GBLK, _EB, _H), jnp.float32),
        pltpu.VMEM((_EB, _H), jnp.float32),
        pltpu.VMEM_SHARED((_NPAD, _H), jnp.float32),
        pltpu.SemaphoreType.DMA,
        pltpu.SemaphoreType.DMA,
        pltpu.SemaphoreType.DMA,
    ],
    compiler_params=pltpu.CompilerParams(use_tc_tiling_on_sc=False),
)
def _edge_agg(src_hbm, dst_hbm, hp_hbm, out_hbm,
              src_v, dst_v, rows_v, zero_v, acc_sh, sem_i, sem_g, sem_s):
    cid = lax.axis_index("c")
    sid = lax.axis_index("s")
    wid = cid * _NS + sid

    def _idx_load(b):
        g0 = wid * _GPW + b * _GBLK
        p = b % 3
        return [
            pltpu.async_copy(src_hbm.at[pl.ds(g0, _GBLK)], src_v.at[p], sem_i),
            pltpu.async_copy(dst_hbm.at[pl.ds(g0, _GBLK)], dst_v.at[p], sem_i),
        ]

    idx_d = {0: _idx_load(0)}

    def _zrow(i, _):
        z16 = jnp.zeros((16,), jnp.float32)
        zero_v[i, 0:16] = z16
        zero_v[i, 16:32] = z16
        return 0
    lax.fori_loop(0, _EB, _zrow, 0)
    zd = [
        pltpu.async_copy(zero_v, acc_sh.at[pl.ds(sid * _RPT + k * _EB, _EB)],
                         sem_s)
        for k in range(_RPT // _EB)
    ]
    for d in zd:
        d.wait()
    plsc.subcore_barrier()

    sca_d = {}
    for b in range(_NBLK):
        if b >= 2:
            for d in sca_d.pop(b - 2):
                d.wait()
        if b + 1 < _NBLK:
            idx_d[b + 1] = _idx_load(b + 1)
        for d in idx_d.pop(b):
            d.wait()
        p3, p2 = b % 3, b % 2
        gat = [
            pltpu.async_copy(hp_hbm.at[src_v.at[p3, j]], rows_v.at[p2, j],
                             sem_g)
            for j in range(_GBLK)
        ]
        for d in gat:
            d.wait()
        sca_d[b] = [
            pltpu.async_copy(rows_v.at[p2, j], acc_sh.at[dst_v.at[p3, j]],
                             sem_s, add=True)
            for j in range(_GBLK)
        ]
    for b in (_NBLK - 2, _NBLK - 1):
        for d in sca_d.pop(b):
            d.wait()

    plsc.subcore_barrier()
    base = cid * _NPAD + sid * _RPT
    outd = [
        pltpu.async_copy(acc_sh.at[pl.ds(sid * _RPT + k * _EB, _EB)],
                         out_hbm.at[pl.ds(base + k * _EB, _EB)], sem_g)
        for k in range(_RPT // _EB)
    ]
    for d in outd:
        d.wait()


def _proj_body(x_ref, w_ref, o_ref):
    o_ref[...] = jnp.dot(x_ref[...], w_ref[...],
                         preferred_element_type=jnp.float32)


def _mid_body(hp_ref, parts_ref, b1_ref, gamma_ref, beta_ref,
              w2_ref, b2_ref, w1n_ref, o_ref):
    z = (hp_ref[...] + parts_ref[0:_N, :] + parts_ref[_NPAD:_NPAD + _N, :]
         + b1_ref[...])
    mean = jnp.mean(z, axis=0, keepdims=True)
    var = jnp.mean((z - mean) ** 2, axis=0, keepdims=True)
    zn = (z - mean) * lax.rsqrt(var + 1e-5) * gamma_ref[...] + beta_ref[...]
    a = jnp.maximum(zn, 0.0)
    h = jnp.maximum(
        jnp.dot(a, w2_ref[...], preferred_element_type=jnp.float32)
        + b2_ref[...], 0.0)
    o_ref[...] = jnp.dot(h, w1n_ref[...], preferred_element_type=jnp.float32)


def _last_body(hp_ref, parts_ref, b1_ref, gamma_ref, beta_ref,
               w2_ref, b2_ref, batch_ref, fw1_ref, fb1_ref,
               fw2_ref, fb2_ref, o_ref):
    z = (hp_ref[...] + parts_ref[0:_N, :] + parts_ref[_NPAD:_NPAD + _N, :]
         + b1_ref[...])
    mean = jnp.mean(z, axis=0, keepdims=True)
    var = jnp.mean((z - mean) ** 2, axis=0, keepdims=True)
    zn = (z - mean) * lax.rsqrt(var + 1e-5) * gamma_ref[...] + beta_ref[...]
    a = jnp.maximum(zn, 0.0)
    h = jnp.maximum(
        jnp.dot(a, w2_ref[...], preferred_element_type=jnp.float32)
        + b2_ref[...], 0.0)
    giota = lax.broadcasted_iota(jnp.int32, (_N, _G), 1)
    onehot = (batch_ref[...] == giota).astype(jnp.float32)
    g = lax.dot_general(onehot, h, (((0,), (0,)), ((), ())),
                        preferred_element_type=jnp.float32)
    g = jnp.maximum(
        jnp.dot(g, fw1_ref[...], preferred_element_type=jnp.float32)
        + fb1_ref[...], 0.0)
    o_ref[...] = (jnp.dot(g, fw2_ref[...], preferred_element_type=jnp.float32)
                  + fb2_ref[...])


_proj = pl.pallas_call(
    _proj_body, out_shape=jax.ShapeDtypeStruct((_N, _H), jnp.float32))

_mid = pl.pallas_call(
    _mid_body, out_shape=jax.ShapeDtypeStruct((_N, _H), jnp.float32))

_last = pl.pallas_call(
    _last_body, out_shape=jax.ShapeDtypeStruct((_G, _C), jnp.float32))


def kernel(x, edge_index, batch, params):
    src = edge_index[0]
    dst = edge_index[1]
    pad = _EPAD - _E
    src_p = jnp.concatenate(
        [src, jnp.zeros((pad,), jnp.int32)]).reshape(_EPAD // _EB, _EB)
    dst_pad = _N + (jnp.arange(pad, dtype=jnp.int32) % (_NPAD - _N))
    dst_p = jnp.concatenate([dst, dst_pad]).reshape(_EPAD // _EB, _EB)

    convs = params['convs']
    row = lambda v: v.reshape(1, -1)

    hp = _proj(x, convs[0]['W1'])
    for l in range(4):
        p = convs[l]
        parts = _edge_agg(src_p, dst_p, hp)
        hp = _mid(hp, parts, row(p['b1']), row(p['gamma']), row(p['beta']),
                  p['W2'], row(p['b2']), convs[l + 1]['W1'])
    p = convs[4]
    parts = _edge_agg(src_p, dst_p, hp)
    out = _last(hp, parts, row(p['b1']), row(p['gamma']), row(p['beta']),
                p['W2'], row(p['b2']), batch.reshape(_N, 1),
                params['fW1'], row(params['fb1']),
                params['fW2'], row(params['fb2']))
    return out


# speedup vs baseline: 9.0565x; 1.0057x over previous
import functools

import jax
import jax.numpy as jnp
from jax import lax
from jax.experimental import pallas as pl
from jax.experimental.pallas import tpu as pltpu
from jax.experimental.pallas import tpu_sc as plsc

_N = 10000
_E = 320000
_F = 128
_H = 32
_C = 10
_G = 64

_NC = 2
_NS = 16
_NW = _NC * _NS

_EB = 1024
_GPW = 10
_EPAD = _NW * _GPW * _EB
_GBLK = 1
_NBLK = _GPW // _GBLK
_ZB = 128
_NPAD = 10240
_RPT = _NPAD // _NS


def _edge_mesh():
    return plsc.VectorSubcoreMesh(
        core_axis_name="c", subcore_axis_name="s",
        num_cores=_NC, num_subcores=_NS)


@functools.partial(
    pl.kernel,
    out_type=jax.ShapeDtypeStruct((_NC * _NPAD, _H), jnp.float32),
    mesh=_edge_mesh(),
    scratch_types=[
        pltpu.VMEM((3, _GBLK, _EB), jnp.int32),
        pltpu.VMEM((3, _GBLK, _EB), jnp.int32),
        pltpu.VMEM((2, _GBLK, _EB, _H), jnp.float32),
        pltpu.VMEM((_ZB, _H), jnp.float32),
        pltpu.VMEM_SHARED((_NPAD, _H), jnp.float32),
        pltpu.SemaphoreType.DMA,
        pltpu.SemaphoreType.DMA,
        pltpu.SemaphoreType.DMA,
    ],
    compiler_params=pltpu.CompilerParams(use_tc_tiling_on_sc=False),
)
def _edge_agg(src_hbm, dst_hbm, hp_hbm, out_hbm,
              src_v, dst_v, rows_v, zero_v, acc_sh, sem_i, sem_g, sem_s):
    cid = lax.axis_index("c")
    sid = lax.axis_index("s")
    wid = cid * _NS + sid

    def _idx_load(b):
        g0 = wid * _GPW + b * _GBLK
        p = b % 3
        return [
            pltpu.async_copy(src_hbm.at[pl.ds(g0, _GBLK)], src_v.at[p], sem_i),
            pltpu.async_copy(dst_hbm.at[pl.ds(g0, _GBLK)], dst_v.at[p], sem_i),
        ]

    idx_d = {0: _idx_load(0)}

    def _zrow(i, _):
        z16 = jnp.zeros((16,), jnp.float32)
        zero_v[i, 0:16] = z16
        zero_v[i, 16:32] = z16
        return 0
    lax.fori_loop(0, _ZB, _zrow, 0)
    zd = [
        pltpu.async_copy(zero_v, acc_sh.at[pl.ds(sid * _RPT + k * _ZB, _ZB)],
                         sem_s)
        for k in range(_RPT // _ZB)
    ]
    for d in zd:
        d.wait()
    plsc.subcore_barrier()

    sca_d = {}
    for b in range(_NBLK):
        if b >= 2:
            for d in sca_d.pop(b - 2):
                d.wait()
        if b + 1 < _NBLK:
            idx_d[b + 1] = _idx_load(b + 1)
        for d in idx_d.pop(b):
            d.wait()
        p3, p2 = b % 3, b % 2
        gat = [
            pltpu.async_copy(hp_hbm.at[src_v.at[p3, j]], rows_v.at[p2, j],
                             sem_g)
            for j in range(_GBLK)
        ]
        for d in gat:
            d.wait()
        sca_d[b] = [
            pltpu.async_copy(rows_v.at[p2, j], acc_sh.at[dst_v.at[p3, j]],
                             sem_s, add=True)
            for j in range(_GBLK)
        ]
    for b in (_NBLK - 2, _NBLK - 1):
        for d in sca_d.pop(b):
            d.wait()

    plsc.subcore_barrier()
    base = cid * _NPAD + sid * _RPT
    outd = [
        pltpu.async_copy(acc_sh.at[pl.ds(sid * _RPT + k * _ZB, _ZB)],
                         out_hbm.at[pl.ds(base + k * _ZB, _ZB)], sem_g)
        for k in range(_RPT // _ZB)
    ]
    for d in outd:
        d.wait()


def _proj_body(x_ref, w_ref, o_ref):
    o_ref[...] = jnp.dot(x_ref[...], w_ref[...],
                         preferred_element_type=jnp.float32)


def _mid_body(hp_ref, parts_ref, b1_ref, gamma_ref, beta_ref,
              w2_ref, b2_ref, w1n_ref, o_ref):
    z = (hp_ref[...] + parts_ref[0:_N, :] + parts_ref[_NPAD:_NPAD + _N, :]
         + b1_ref[...])
    mean = jnp.mean(z, axis=0, keepdims=True)
    var = jnp.mean((z - mean) ** 2, axis=0, keepdims=True)
    zn = (z - mean) * lax.rsqrt(var + 1e-5) * gamma_ref[...] + beta_ref[...]
    a = jnp.maximum(zn, 0.0)
    h = jnp.maximum(
        jnp.dot(a, w2_ref[...], preferred_element_type=jnp.float32)
        + b2_ref[...], 0.0)
    o_ref[...] = jnp.dot(h, w1n_ref[...], preferred_element_type=jnp.float32)


def _last_body(hp_ref, parts_ref, b1_ref, gamma_ref, beta_ref,
               w2_ref, b2_ref, batch_ref, fw1_ref, fb1_ref,
               fw2_ref, fb2_ref, o_ref):
    z = (hp_ref[...] + parts_ref[0:_N, :] + parts_ref[_NPAD:_NPAD + _N, :]
         + b1_ref[...])
    mean = jnp.mean(z, axis=0, keepdims=True)
    var = jnp.mean((z - mean) ** 2, axis=0, keepdims=True)
    zn = (z - mean) * lax.rsqrt(var + 1e-5) * gamma_ref[...] + beta_ref[...]
    a = jnp.maximum(zn, 0.0)
    h = jnp.maximum(
        jnp.dot(a, w2_ref[...], preferred_element_type=jnp.float32)
        + b2_ref[...], 0.0)
    giota = lax.broadcasted_iota(jnp.int32, (_N, _G), 1)
    onehot = (batch_ref[...] == giota).astype(jnp.float32)
    g = lax.dot_general(onehot, h, (((0,), (0,)), ((), ())),
                        preferred_element_type=jnp.float32)
    g = jnp.maximum(
        jnp.dot(g, fw1_ref[...], preferred_element_type=jnp.float32)
        + fb1_ref[...], 0.0)
    o_ref[...] = (jnp.dot(g, fw2_ref[...], preferred_element_type=jnp.float32)
                  + fb2_ref[...])


_proj = pl.pallas_call(
    _proj_body, out_shape=jax.ShapeDtypeStruct((_N, _H), jnp.float32))

_mid = pl.pallas_call(
    _mid_body, out_shape=jax.ShapeDtypeStruct((_N, _H), jnp.float32))

_last = pl.pallas_call(
    _last_body, out_shape=jax.ShapeDtypeStruct((_G, _C), jnp.float32))


def kernel(x, edge_index, batch, params):
    src = edge_index[0]
    dst = edge_index[1]
    pad = _EPAD - _E
    src_p = jnp.concatenate(
        [src, jnp.zeros((pad,), jnp.int32)]).reshape(_EPAD // _EB, _EB)
    dst_pad = _N + (jnp.arange(pad, dtype=jnp.int32) % (_NPAD - _N))
    dst_p = jnp.concatenate([dst, dst_pad]).reshape(_EPAD // _EB, _EB)

    convs = params['convs']
    row = lambda v: v.reshape(1, -1)

    hp = _proj(x, convs[0]['W1'])
    for l in range(4):
        p = convs[l]
        parts = _edge_agg(src_p, dst_p, hp)
        hp = _mid(hp, parts, row(p['b1']), row(p['gamma']), row(p['beta']),
                  p['W2'], row(p['b2']), convs[l + 1]['W1'])
    p = convs[4]
    parts = _edge_agg(src_p, dst_p, hp)
    out = _last(hp, parts, row(p['b1']), row(p['gamma']), row(p['beta']),
                p['W2'], row(p['b2']), batch.reshape(_N, 1),
                params['fW1'], row(params['fb1']),
                params['fW2'], row(params['fb2']))
    return out


# speedup vs baseline: 10.6177x; 1.1724x over previous
import functools

import jax
import jax.numpy as jnp
from jax import lax
from jax.experimental import pallas as pl
from jax.experimental.pallas import tpu as pltpu
from jax.experimental.pallas import tpu_sc as plsc

_N = 10000
_E = 320000
_F = 128
_H = 32
_C = 10
_G = 64

_NC = 2
_NS = 16
_NW = _NC * _NS

_EB = 1024
_EPAD = 327680
_NGRP = _EPAD // _EB
_GPW = _NGRP // _NS
_GBLK = 1
_NBLK = _GPW // _GBLK
_HH = _H // 2
_ZB = 128
_NPAD = 10240
_RPT = _NPAD // _NS


def _edge_mesh():
    return plsc.VectorSubcoreMesh(
        core_axis_name="c", subcore_axis_name="s",
        num_cores=_NC, num_subcores=_NS)


@functools.partial(
    pl.kernel,
    out_type=jax.ShapeDtypeStruct((_NC * _NPAD, _HH), jnp.float32),
    mesh=_edge_mesh(),
    scratch_types=[
        pltpu.VMEM((3, _GBLK, _EB), jnp.int32),
        pltpu.VMEM((3, _GBLK, _EB), jnp.int32),
        pltpu.VMEM((2, _GBLK, _EB, _HH), jnp.float32),
        pltpu.VMEM((_ZB, _HH), jnp.float32),
        pltpu.VMEM_SHARED((_NPAD, _HH), jnp.float32),
        pltpu.SemaphoreType.DMA,
        pltpu.SemaphoreType.DMA,
        pltpu.SemaphoreType.DMA,
    ],
    compiler_params=pltpu.CompilerParams(use_tc_tiling_on_sc=False),
)
def _edge_agg(src_hbm, dst_hbm, hp_hbm, out_hbm,
              src_v, dst_v, rows_v, zero_v, acc_sh, sem_i, sem_g, sem_s):
    cid = lax.axis_index("c")
    sid = lax.axis_index("s")
    wid = cid * _NS + sid

    def _idx_load(b):
        g0 = sid * _GPW + b * _GBLK
        p = b % 3
        return [
            pltpu.async_copy(src_hbm.at[pl.ds(g0, _GBLK)], src_v.at[p], sem_i),
            pltpu.async_copy(dst_hbm.at[pl.ds(g0, _GBLK)], dst_v.at[p], sem_i),
        ]

    idx_d = {0: _idx_load(0)}

    def _zrow(i, _):
        zero_v[i, 0:16] = jnp.zeros((16,), jnp.float32)
        return 0
    lax.fori_loop(0, _ZB, _zrow, 0)
    zd = [
        pltpu.async_copy(zero_v, acc_sh.at[pl.ds(sid * _RPT + k * _ZB, _ZB)],
                         sem_s)
        for k in range(_RPT // _ZB)
    ]
    for d in zd:
        d.wait()
    plsc.subcore_barrier()

    sca_d = {}
    for b in range(_NBLK):
        if b >= 2:
            for d in sca_d.pop(b - 2):
                d.wait()
        if b + 1 < _NBLK:
            idx_d[b + 1] = _idx_load(b + 1)
        for d in idx_d.pop(b):
            d.wait()
        p3, p2 = b % 3, b % 2
        def _off(k, _):
            sl = pl.ds(k * 16, 16)
            src_v[p3, 0, sl] = src_v[p3, 0, sl] + cid * _N
            return 0
        lax.fori_loop(0, _EB // 16, _off, 0)
        gat = [
            pltpu.async_copy(hp_hbm.at[src_v.at[p3, j]], rows_v.at[p2, j],
                             sem_g)
            for j in range(_GBLK)
        ]
        for d in gat:
            d.wait()
        sca_d[b] = [
            pltpu.async_copy(rows_v.at[p2, j], acc_sh.at[dst_v.at[p3, j]],
                             sem_s, add=True)
            for j in range(_GBLK)
        ]
    for b in (_NBLK - 2, _NBLK - 1):
        for d in sca_d.pop(b):
            d.wait()

    plsc.subcore_barrier()
    base = cid * _NPAD + sid * _RPT
    outd = [
        pltpu.async_copy(acc_sh.at[pl.ds(sid * _RPT + k * _ZB, _ZB)],
                         out_hbm.at[pl.ds(base + k * _ZB, _ZB)], sem_g)
        for k in range(_RPT // _ZB)
    ]
    for d in outd:
        d.wait()


def _split_store(o_ref, h):
    o_ref[0:_N, :] = h[:, 0:_HH]
    o_ref[_N:2 * _N, :] = h[:, _HH:_H]


def _merge(ref):
    return jnp.concatenate([ref[0:_N, :], ref[_N:2 * _N, :]], axis=1)


def _proj_body(x_ref, w_ref, o_ref):
    _split_store(o_ref, jnp.dot(x_ref[...], w_ref[...],
                                preferred_element_type=jnp.float32))


def _mid_body(hp_ref, parts_ref, b1_ref, gamma_ref, beta_ref,
              w2_ref, b2_ref, w1n_ref, o_ref):
    agg = jnp.concatenate([parts_ref[0:_N, :],
                           parts_ref[_NPAD:_NPAD + _N, :]], axis=1)
    z = _merge(hp_ref) + agg + b1_ref[...]
    mean = jnp.mean(z, axis=0, keepdims=True)
    var = jnp.mean((z - mean) ** 2, axis=0, keepdims=True)
    zn = (z - mean) * lax.rsqrt(var + 1e-5) * gamma_ref[...] + beta_ref[...]
    a = jnp.maximum(zn, 0.0)
    h = jnp.maximum(
        jnp.dot(a, w2_ref[...], preferred_element_type=jnp.float32)
        + b2_ref[...], 0.0)
    _split_store(o_ref, jnp.dot(h, w1n_ref[...],
                                 preferred_element_type=jnp.float32))


def _last_body(hp_ref, parts_ref, b1_ref, gamma_ref, beta_ref,
               w2_ref, b2_ref, batch_ref, fw1_ref, fb1_ref,
               fw2_ref, fb2_ref, o_ref):
    agg = jnp.concatenate([parts_ref[0:_N, :],
                           parts_ref[_NPAD:_NPAD + _N, :]], axis=1)
    z = _merge(hp_ref) + agg + b1_ref[...]
    mean = jnp.mean(z, axis=0, keepdims=True)
    var = jnp.mean((z - mean) ** 2, axis=0, keepdims=True)
    zn = (z - mean) * lax.rsqrt(var + 1e-5) * gamma_ref[...] + beta_ref[...]
    a = jnp.maximum(zn, 0.0)
    h = jnp.maximum(
        jnp.dot(a, w2_ref[...], preferred_element_type=jnp.float32)
        + b2_ref[...], 0.0)
    giota = lax.broadcasted_iota(jnp.int32, (_N, _G), 1)
    onehot = (batch_ref[...] == giota).astype(jnp.float32)
    g = lax.dot_general(onehot, h, (((0,), (0,)), ((), ())),
                        preferred_element_type=jnp.float32)
    g = jnp.maximum(
        jnp.dot(g, fw1_ref[...], preferred_element_type=jnp.float32)
        + fb1_ref[...], 0.0)
    o_ref[...] = (jnp.dot(g, fw2_ref[...], preferred_element_type=jnp.float32)
                  + fb2_ref[...])


_proj = pl.pallas_call(
    _proj_body, out_shape=jax.ShapeDtypeStruct((2 * _N, _HH), jnp.float32))

_mid = pl.pallas_call(
    _mid_body, out_shape=jax.ShapeDtypeStruct((2 * _N, _HH), jnp.float32))

_last = pl.pallas_call(
    _last_body, out_shape=jax.ShapeDtypeStruct((_G, _C), jnp.float32))


def kernel(x, edge_index, batch, params):
    src = edge_index[0]
    dst = edge_index[1]
    pad = _EPAD - _E
    src_p = jnp.concatenate(
        [src, jnp.zeros((pad,), jnp.int32)]).reshape(_EPAD // _EB, _EB)
    dst_pad = _N + (jnp.arange(pad, dtype=jnp.int32) % (_NPAD - _N))
    dst_p = jnp.concatenate([dst, dst_pad]).reshape(_EPAD // _EB, _EB)

    convs = params['convs']
    row = lambda v: v.reshape(1, -1)

    hp = _proj(x, convs[0]['W1'])
    for l in range(4):
        p = convs[l]
        parts = _edge_agg(src_p, dst_p, hp)
        hp = _mid(hp, parts, row(p['b1']), row(p['gamma']), row(p['beta']),
                  p['W2'], row(p['b2']), convs[l + 1]['W1'])
    p = convs[4]
    parts = _edge_agg(src_p, dst_p, hp)
    out = _last(hp, parts, row(p['b1']), row(p['gamma']), row(p['beta']),
                p['W2'], row(p['b2']), batch.reshape(_N, 1),
                params['fW1'], row(params['fb1']),
                params['fW2'], row(params['fb2']))
    return out
